# baseline (speedup 1.0000x reference)
import jax
import jax.numpy as jnp
from jax import lax
from jax.experimental import pallas as pl
from jax.experimental.pallas import tpu as pltpu
from jax.experimental.pallas import tpu_sc as plsc

N = 10000
D = 128
E = 320000

NC = 2
NS = 16
NPAD = 10240
HALF = NPAD // NC
AROWS = HALF + NS
K = 80
NBUF = 4
NCHUNK = 252
E_TILE = NCHUNK * K
ROWS_PER_TILE = HALF // NS
DST_PAD = 1 << 29


def _seg_body(x_hbm, src_hbm, dst_hbm, out_hbm,
              src_v, d0, d1, d2, d3, r0, r1, r2, r3, agg, gsem, dsem):
    cid = lax.axis_index("c")
    sid = lax.axis_index("s")
    dv = [d0, d1, d2, d3]
    rv = [r0, r1, r2, r3]

    z = jnp.zeros((16,), jnp.float32)

    def zrow(r, carry):
        for j in range(D // 16):
            r0[r, pl.ds(j * 16, 16)] = z
        return carry

    lax.fori_loop(0, K, zrow, 0)
    row0 = sid * ROWS_PER_TILE
    for t in range(ROWS_PER_TILE // K):
        pltpu.sync_copy(r0, agg.at[pl.ds(row0 + t * K, K)])
    plsc.subcore_barrier()

    pltpu.sync_copy(src_hbm.at[sid], src_v)

    base = cid * HALF
    trash = jnp.full((16,), HALF, jnp.int32) + sid

    def start(c, b):
        pltpu.async_copy(x_hbm.at[src_v.at[c]], rv[b], gsem)
        pltpu.async_copy(dst_hbm.at[sid, c], dv[b], dsem)

    def finish(c, b):
        pltpu.make_async_copy(dst_hbm.at[sid, c], dv[b], dsem).wait()
        for j in range(K // 16):
            v = dv[b][pl.ds(j * 16, 16)] - base
            ok = (v >= 0) & (v < HALF)
            dv[b][pl.ds(j * 16, 16)] = jnp.where(ok, v, trash)
        pltpu.make_async_copy(x_hbm.at[src_v.at[c]], rv[b], gsem).wait()

        @pl.when(c + (NBUF - 1) < NCHUNK)
        def _():
            start(c + (NBUF - 1), (b + NBUF - 1) % NBUF)

        pltpu.sync_copy(rv[b], agg.at[dv[b]], add=True)

    for b in range(NBUF - 1):
        start(b, b)

    def ring(t, carry):
        for b in range(NBUF):
            finish(t * NBUF + b, b)
        return carry

    lax.fori_loop(0, NCHUNK // NBUF, ring, 0)
    plsc.subcore_barrier()

    pltpu.sync_copy(agg.at[pl.ds(row0, ROWS_PER_TILE)],
                    out_hbm.at[pl.ds(base + row0, ROWS_PER_TILE)])


@jax.jit
def _seg_sum(x, src_r, dst_r):
    mesh = plsc.VectorSubcoreMesh(core_axis_name="c", subcore_axis_name="s")
    return pl.kernel(
        _seg_body,
        out_type=jax.ShapeDtypeStruct((NPAD, D), jnp.float32),
        mesh=mesh,
        scratch_types=[
            pltpu.VMEM((NCHUNK, K), jnp.int32),
            pltpu.VMEM((K,), jnp.int32),
            pltpu.VMEM((K,), jnp.int32),
            pltpu.VMEM((K,), jnp.int32),
            pltpu.VMEM((K,), jnp.int32),
            pltpu.VMEM((K, D), jnp.float32),
            pltpu.VMEM((K, D), jnp.float32),
            pltpu.VMEM((K, D), jnp.float32),
            pltpu.VMEM((K, D), jnp.float32),
            pltpu.VMEM_SHARED((AROWS, D), jnp.float32),
            pltpu.SemaphoreType.DMA,
            pltpu.SemaphoreType.DMA,
        ],
    )(x, src_r, dst_r)


def _mlp_body(p_ref, w_ref, b_ref, o_ref):
    y = jnp.dot(p_ref[...], w_ref[...],
                preferred_element_type=jnp.float32) + b_ref[...]
    o_ref[...] = jnp.maximum(y, 0.0)


@jax.jit
def _mlp(p, W, b):
    R = 1024
    return pl.pallas_call(
        _mlp_body,
        grid=(NPAD // R,),
        in_specs=[
            pl.BlockSpec((R, D), lambda i: (i, 0)),
            pl.BlockSpec((D, D), lambda i: (0, 0)),
            pl.BlockSpec((1, D), lambda i: (0, 0)),
        ],
        out_specs=pl.BlockSpec((R, D), lambda i: (i, 0)),
        out_shape=jax.ShapeDtypeStruct((NPAD, D), jnp.float32),
    )(p, W, b.reshape(1, D))


def _pad_idx(a, fill):
    a = a.reshape(NS, E // NS)
    a = jnp.pad(a, ((0, 0), (0, E_TILE - E // NS)), constant_values=fill)
    return a.reshape(NS, NCHUNK, K)


def kernel(h, edge_index, W1, b1, W2, b2):
    src_r = _pad_idx(edge_index[0], 0)
    dst_r = _pad_idx(edge_index[1], DST_PAD)
    a1 = _seg_sum(h, src_r, dst_r)
    x = _mlp(a1, W1, b1)
    a2 = _seg_sum(x, src_r, dst_r)
    return _mlp(a2, W2, b2)[:N]
